# baseline (speedup 1.0000x reference)
import functools

import jax
import jax.numpy as jnp
from jax import lax
from jax.experimental import pallas as pl
from jax.experimental.pallas import tpu as pltpu
from jax.experimental.pallas import tpu_sc as plsc

VOCAB = 1000000
DIM = 32
B = 1024
K = 20
QW = 128
RPQ = QW // DIM

NC = 2
NS = 16
NW = NC * NS
BPW = B // NW
NPW = B * K // NW
NCHUNK = NPW // 128


def _sc_gather_body(cidx_hbm, tidx_hbm, nidx_hbm, emb_hbm,
                    outc_hbm, outt_hbm, outn_hbm,
                    idx_c, idx_t, idx_n, rows_c, rows_t, rows_n, sem):
    w = lax.axis_index("s") * NC + lax.axis_index("c")
    pltpu.sync_copy(cidx_hbm.at[pl.ds(w * BPW, BPW)], idx_c)
    pltpu.sync_copy(tidx_hbm.at[pl.ds(w * BPW, BPW)], idx_t)
    for j in range(NCHUNK):
        pltpu.sync_copy(
            nidx_hbm.at[pl.ds(w * NPW + j * 128, 128)], idx_n[j])
    cps = [
        pltpu.async_copy(emb_hbm.at[idx_c], rows_c, sem),
        pltpu.async_copy(emb_hbm.at[idx_t], rows_t, sem),
    ]
    for j in range(NCHUNK):
        cps.append(pltpu.async_copy(
            emb_hbm.at[idx_n[j]],
            rows_n.at[pl.ds(j * 128, 128)], sem))
    for cp in cps:
        cp.wait()
    pltpu.sync_copy(rows_c, outc_hbm.at[pl.ds(w * BPW, BPW)])
    pltpu.sync_copy(rows_t, outt_hbm.at[pl.ds(w * BPW, BPW)])
    pltpu.sync_copy(rows_n, outn_hbm.at[pl.ds(w * NPW, NPW)])


_sc_gather = functools.partial(
    pl.kernel,
    out_type=(
        jax.ShapeDtypeStruct((B, QW), jnp.float32),
        jax.ShapeDtypeStruct((B, QW), jnp.float32),
        jax.ShapeDtypeStruct((B * K, QW), jnp.float32),
    ),
    mesh=plsc.VectorSubcoreMesh(core_axis_name="c", subcore_axis_name="s"),
    scratch_types=[
        pltpu.VMEM((BPW,), jnp.int32),
        pltpu.VMEM((BPW,), jnp.int32),
        [pltpu.VMEM((128,), jnp.int32) for _ in range(NCHUNK)],
        pltpu.VMEM((BPW, QW), jnp.float32),
        pltpu.VMEM((BPW, QW), jnp.float32),
        pltpu.VMEM((NPW, QW), jnp.float32),
        pltpu.SemaphoreType.DMA,
    ],
)(_sc_gather_body)


def _select_strip(rows, strip):
    out = jnp.zeros((rows.shape[0], DIM), jnp.float32)
    for s in range(RPQ):
        m = (strip == s).astype(jnp.float32)
        out = out + m * rows[:, s * DIM:(s + 1) * DIM]
    return out


def _tc_loss_body(c_ref, t_ref, n_ref, cs_ref, ts_ref, ns_ref, out_ref):
    c = _select_strip(c_ref[...], cs_ref[...])
    t = _select_strip(t_ref[...], ts_ref[...])
    nsum = jnp.zeros((B, DIM), jnp.float32)
    for k in range(K):
        nsum = nsum + _select_strip(
            n_ref[pl.ds(k * B, B), :], ns_ref[pl.ds(k * B, B), :])
    p = jnp.sum(t * c, axis=1, keepdims=True)
    n = -jnp.sum(nsum * c, axis=1, keepdims=True)
    loss = jax.nn.log_sigmoid(p) + jax.nn.log_sigmoid(n)
    out_ref[0, 0] = -jnp.mean(loss)


def kernel(center, target, noise, embeddings):
    center = center.astype(jnp.int32)
    target = target.astype(jnp.int32)
    nidx = jnp.transpose(noise.astype(jnp.int32)).reshape(B * K)
    one = (center[0] * 0 + 1).astype(jnp.float32)
    emb_q = (embeddings * one).reshape(VOCAB // RPQ, QW)
    c_rows, t_rows, n_rows = _sc_gather(
        center // RPQ, target // RPQ, nidx // RPQ, emb_q)
    out = pl.pallas_call(
        _tc_loss_body,
        out_shape=jax.ShapeDtypeStruct((1, 1), jnp.float32),
        out_specs=pl.BlockSpec(memory_space=pltpu.SMEM),
    )(c_rows, t_rows, n_rows,
      (center % RPQ)[:, None], (target % RPQ)[:, None],
      (nidx % RPQ)[:, None])
    return out[0, 0]


# speedup vs baseline: 1.0392x; 1.0392x over previous
import functools

import jax
import jax.numpy as jnp
from jax import lax
from jax.experimental import pallas as pl
from jax.experimental.pallas import tpu as pltpu
from jax.experimental.pallas import tpu_sc as plsc

VOCAB = 1000000
DIM = 32
B = 1024
K = 20

NC = 2
NS = 16
NW = NC * NS
BPW = B // NW
NPW = B * K // NW
NCHUNK = NPW // 128


def _sc_gather_body(cidx_hbm, tidx_hbm, nidx_hbm, emb_hbm,
                    outc_hbm, outt_hbm, outn_hbm,
                    idx_c, idx_t, idx_n, rows_c, rows_t, rows_n, sem):
    w = lax.axis_index("s") * NC + lax.axis_index("c")
    pltpu.sync_copy(cidx_hbm.at[pl.ds(w * BPW, BPW)], idx_c)
    pltpu.sync_copy(tidx_hbm.at[pl.ds(w * BPW, BPW)], idx_t)
    for j in range(NCHUNK):
        pltpu.sync_copy(
            nidx_hbm.at[pl.ds(w * NPW + j * 128, 128)], idx_n[j])
    cps = [
        pltpu.async_copy(emb_hbm.at[idx_c], rows_c, sem),
        pltpu.async_copy(emb_hbm.at[idx_t], rows_t, sem),
    ]
    for j in range(NCHUNK):
        cps.append(pltpu.async_copy(
            emb_hbm.at[idx_n[j]],
            rows_n.at[pl.ds(j * 128, 128)], sem))
    for cp in cps:
        cp.wait()
    pltpu.sync_copy(rows_c, outc_hbm.at[pl.ds(w * BPW, BPW)])
    pltpu.sync_copy(rows_t, outt_hbm.at[pl.ds(w * BPW, BPW)])
    pltpu.sync_copy(rows_n, outn_hbm.at[pl.ds(w * NPW, NPW)])


_sc_gather = functools.partial(
    pl.kernel,
    out_type=(
        jax.ShapeDtypeStruct((B, DIM), jnp.float32),
        jax.ShapeDtypeStruct((B, DIM), jnp.float32),
        jax.ShapeDtypeStruct((B * K, DIM), jnp.float32),
    ),
    mesh=plsc.VectorSubcoreMesh(core_axis_name="c", subcore_axis_name="s"),
    compiler_params=pltpu.CompilerParams(use_tc_tiling_on_sc=False),
    scratch_types=[
        pltpu.VMEM((BPW,), jnp.int32),
        pltpu.VMEM((BPW,), jnp.int32),
        [pltpu.VMEM((128,), jnp.int32) for _ in range(NCHUNK)],
        pltpu.VMEM((BPW, DIM), jnp.float32),
        pltpu.VMEM((BPW, DIM), jnp.float32),
        pltpu.VMEM((NPW, DIM), jnp.float32),
        pltpu.SemaphoreType.DMA,
    ],
)(_sc_gather_body)


def _tc_loss_body(c_ref, t_ref, n_ref, out_ref):
    c = c_ref[...]
    t = t_ref[...]
    nsum = jnp.sum(n_ref[...].reshape(B, K, DIM), axis=1)
    p = jnp.sum(t * c, axis=1, keepdims=True)
    n = -jnp.sum(nsum * c, axis=1, keepdims=True)
    loss = jax.nn.log_sigmoid(p) + jax.nn.log_sigmoid(n)
    out_ref[0, 0] = -jnp.mean(loss)


def kernel(center, target, noise, embeddings):
    center = center.astype(jnp.int32)
    target = target.astype(jnp.int32)
    nidx = noise.astype(jnp.int32).reshape(B * K)
    c_rows, t_rows, n_rows = _sc_gather(center, target, nidx, embeddings)
    out = pl.pallas_call(
        _tc_loss_body,
        out_shape=jax.ShapeDtypeStruct((1, 1), jnp.float32),
        out_specs=pl.BlockSpec(memory_space=pltpu.SMEM),
    )(c_rows, t_rows, n_rows)
    return out[0, 0]


# speedup vs baseline: 1.5618x; 1.5028x over previous
import functools

import jax
import jax.numpy as jnp
from jax import lax
from jax.experimental import pallas as pl
from jax.experimental.pallas import tpu as pltpu
from jax.experimental.pallas import tpu_sc as plsc

VOCAB = 1000000
DIM = 32
B = 1024
K = 20

NC = 2
NS = 16
NW = NC * NS
BPW = B // NW
NPW = B * K // NW
IPW = 2 * BPW + NPW
G = 16
NG = IPW // G


def _sc_gather_body(cidx_hbm, tidx_hbm, nidx_hbm, emb_hbm,
                    outc_hbm, outt_hbm, outn_hbm,
                    idx_all, buf, rows, sem):
    w = lax.axis_index("s") * NC + lax.axis_index("c")
    pltpu.sync_copy(cidx_hbm.at[pl.ds(w * BPW, BPW)], idx_all.at[pl.ds(0, BPW)])
    pltpu.sync_copy(tidx_hbm.at[pl.ds(w * BPW, BPW)],
                    idx_all.at[pl.ds(BPW, BPW)])
    for j in range(NPW // 128):
        pltpu.sync_copy(nidx_hbm.at[pl.ds(w * NPW + j * 128, 128)],
                        idx_all.at[pl.ds(2 * BPW + j * 128, 128)])

    def fire(g):
        slotbase = (g % 2) * G
        vvec = idx_all[pl.ds(g * G, G)]
        for j in range(G):
            v = vvec[j]
            v8 = pl.multiple_of((v // 8) * 8, 8)
            pltpu.async_copy(
                emb_hbm.at[pl.ds(v8, 8), :], buf.at[slotbase + j], sem)

    fire(0)

    def step(i, _):
        @pl.when(i < NG - 1)
        def _():
            fire(i + 1)
        slotbase = (i % 2) * G
        vvec = idx_all[pl.ds(i * G, G)]
        for j in range(G):
            pltpu.make_async_copy(
                emb_hbm.at[pl.ds(0, 8), :], buf.at[slotbase + j], sem).wait()
            v = vvec[j]
            r = v - (v // 8) * 8
            pos = i * G + j
            rows[pos, pl.ds(0, 16)] = buf[slotbase + j, r, pl.ds(0, 16)]
            rows[pos, pl.ds(16, 16)] = buf[slotbase + j, r, pl.ds(16, 16)]
        return 0

    lax.fori_loop(0, NG, step, 0)

    pltpu.sync_copy(rows.at[pl.ds(0, BPW)], outc_hbm.at[pl.ds(w * BPW, BPW)])
    pltpu.sync_copy(rows.at[pl.ds(BPW, BPW)],
                    outt_hbm.at[pl.ds(w * BPW, BPW)])
    pltpu.sync_copy(rows.at[pl.ds(2 * BPW, NPW)],
                    outn_hbm.at[pl.ds(w * NPW, NPW)])


_sc_gather = functools.partial(
    pl.kernel,
    out_type=(
        jax.ShapeDtypeStruct((B, DIM), jnp.float32),
        jax.ShapeDtypeStruct((B, DIM), jnp.float32),
        jax.ShapeDtypeStruct((B * K, DIM), jnp.float32),
    ),
    mesh=plsc.VectorSubcoreMesh(core_axis_name="c", subcore_axis_name="s"),
    compiler_params=pltpu.CompilerParams(use_tc_tiling_on_sc=True),
    scratch_types=[
        pltpu.VMEM((IPW,), jnp.int32),
        pltpu.VMEM((2 * G, 8, DIM), jnp.float32),
        pltpu.VMEM((IPW, DIM), jnp.float32),
        pltpu.SemaphoreType.DMA,
    ],
)(_sc_gather_body)


def _tc_loss_body(c_ref, t_ref, n_ref, out_ref):
    c = c_ref[...]
    t = t_ref[...]
    nsum = jnp.sum(n_ref[...].reshape(B, K, DIM), axis=1)
    p = jnp.sum(t * c, axis=1, keepdims=True)
    n = -jnp.sum(nsum * c, axis=1, keepdims=True)
    loss = jax.nn.log_sigmoid(p) + jax.nn.log_sigmoid(n)
    out_ref[0, 0] = -jnp.mean(loss)


def kernel(center, target, noise, embeddings):
    center = center.astype(jnp.int32)
    target = target.astype(jnp.int32)
    nidx = noise.astype(jnp.int32).reshape(B * K)
    c_rows, t_rows, n_rows = _sc_gather(center, target, nidx, embeddings)
    out = pl.pallas_call(
        _tc_loss_body,
        out_shape=jax.ShapeDtypeStruct((1, 1), jnp.float32),
        out_specs=pl.BlockSpec(memory_space=pltpu.SMEM),
    )(c_rows, t_rows, n_rows)
    return out[0, 0]


# speedup vs baseline: 2.0127x; 1.2887x over previous
import functools

import jax
import jax.numpy as jnp
from jax import lax
from jax.experimental import pallas as pl
from jax.experimental.pallas import tpu as pltpu
from jax.experimental.pallas import tpu_sc as plsc

VOCAB = 1000000
DIM = 32
B = 1024
K = 20

NC = 2
NS = 16
NW = NC * NS
BPW = B // NW
NPW = B * K // NW
IPW = 2 * BPW + NPW
G = 16
NG = IPW // G


def _sc_gather_body(cidx_hbm, tidx_hbm, nidx_hbm, emb_hbm,
                    outc_hbm, outt_hbm, outn_hbm,
                    idx_all, buf, rows, sem):
    w = lax.axis_index("s") * NC + lax.axis_index("c")
    pltpu.sync_copy(cidx_hbm.at[pl.ds(w * BPW, BPW)], idx_all.at[pl.ds(0, BPW)])
    pltpu.sync_copy(tidx_hbm.at[pl.ds(w * BPW, BPW)],
                    idx_all.at[pl.ds(BPW, BPW)])
    for j in range(NPW // 128):
        pltpu.sync_copy(nidx_hbm.at[pl.ds(w * NPW + j * 128, 128)],
                        idx_all.at[pl.ds(2 * BPW + j * 128, 128)])

    def fire(g):
        slotbase = (g % 2) * G
        vvec = idx_all[pl.ds(g * G, G)]
        for j in range(G):
            v = vvec[j]
            v8 = pl.multiple_of((v // 8) * 8, 8)
            pltpu.async_copy(
                emb_hbm.at[pl.ds(v8, 8), :], buf.at[slotbase + j], sem)

    fire(0)

    def step(i, _):
        @pl.when(i < NG - 1)
        def _():
            fire(i + 1)
        slotbase = (i % 2) * G
        vvec = idx_all[pl.ds(i * G, G)]
        for j in range(G):
            pltpu.make_async_copy(
                emb_hbm.at[pl.ds(0, 8), :], buf.at[slotbase + j], sem).wait()
            v = vvec[j]
            r = v - (v // 8) * 8
            pos = i * G + j
            rows[pos, pl.ds(0, 16)] = buf[slotbase + j, r, pl.ds(0, 16)]
            rows[pos, pl.ds(16, 16)] = buf[slotbase + j, r, pl.ds(16, 16)]
        return 0

    lax.fori_loop(0, NG, step, 0)

    pltpu.sync_copy(rows.at[pl.ds(0, BPW)], outc_hbm.at[pl.ds(w * BPW, BPW)])
    pltpu.sync_copy(rows.at[pl.ds(BPW, BPW)],
                    outt_hbm.at[pl.ds(w * BPW, BPW)])
    pltpu.sync_copy(rows.at[pl.ds(2 * BPW, NPW)],
                    outn_hbm.at[pl.ds(w * NPW, NPW)])


_sc_gather = functools.partial(
    pl.kernel,
    out_type=(
        jax.ShapeDtypeStruct((B, DIM), jnp.float32),
        jax.ShapeDtypeStruct((B, DIM), jnp.float32),
        jax.ShapeDtypeStruct((B * K, DIM), jnp.float32),
    ),
    mesh=plsc.VectorSubcoreMesh(core_axis_name="c", subcore_axis_name="s"),
    compiler_params=pltpu.CompilerParams(use_tc_tiling_on_sc=True),
    scratch_types=[
        pltpu.VMEM((IPW,), jnp.int32),
        pltpu.VMEM((2 * G, 8, DIM), jnp.float32),
        pltpu.VMEM((IPW, DIM), jnp.float32),
        pltpu.SemaphoreType.DMA,
    ],
)(_sc_gather_body)


TW = 16384


def _tc_transpose_body(in_ref, out_ref):
    out_ref[...] = in_ref[...].T


def _tc_transpose(embT):
    grid = (VOCAB + TW - 1) // TW
    return pl.pallas_call(
        _tc_transpose_body,
        grid=(grid,),
        in_specs=[pl.BlockSpec((DIM, TW), lambda c: (0, c))],
        out_specs=pl.BlockSpec((TW, DIM), lambda c: (c, 0)),
        out_shape=jax.ShapeDtypeStruct((VOCAB, DIM), jnp.float32),
    )(embT)


def _tc_loss_body(c_ref, t_ref, n_ref, out_ref):
    c = c_ref[...]
    t = t_ref[...]
    nsum = jnp.sum(n_ref[...].reshape(B, K, DIM), axis=1)
    p = jnp.sum(t * c, axis=1, keepdims=True)
    n = -jnp.sum(nsum * c, axis=1, keepdims=True)
    loss = jax.nn.log_sigmoid(p) + jax.nn.log_sigmoid(n)
    out_ref[0, 0] = -jnp.mean(loss)


def kernel(center, target, noise, embeddings):
    center = center.astype(jnp.int32)
    target = target.astype(jnp.int32)
    nidx = noise.astype(jnp.int32).reshape(B * K)
    emb_rm = _tc_transpose(jnp.transpose(embeddings))
    c_rows, t_rows, n_rows = _sc_gather(center, target, nidx, emb_rm)
    out = pl.pallas_call(
        _tc_loss_body,
        out_shape=jax.ShapeDtypeStruct((1, 1), jnp.float32),
        out_specs=pl.BlockSpec(memory_space=pltpu.SMEM),
    )(c_rows, t_rows, n_rows)
    return out[0, 0]


# speedup vs baseline: 2.0375x; 1.0123x over previous
import functools

import jax
import jax.numpy as jnp
from jax import lax
from jax.experimental import pallas as pl
from jax.experimental.pallas import tpu as pltpu
from jax.experimental.pallas import tpu_sc as plsc

VOCAB = 1000000
DIM = 32
B = 1024
K = 20
RW = 128

NC = 2
NS = 16
NW = NC * NS
BPW = B // NW
NPW = B * K // NW
NCHUNK = NPW // 128

TW = 8192


def _tc_transpose_body(in_ref, out_ref):
    out_ref[:, pl.ds(0, DIM)] = in_ref[...].T


def _tc_transpose(embT):
    grid = (VOCAB + TW - 1) // TW
    return pl.pallas_call(
        _tc_transpose_body,
        grid=(grid,),
        in_specs=[pl.BlockSpec((DIM, TW), lambda c: (0, c))],
        out_specs=pl.BlockSpec((TW, RW), lambda c: (c, 0)),
        out_shape=jax.ShapeDtypeStruct((VOCAB, RW), jnp.float32),
    )(embT)


def _sc_gather_body(cidx_hbm, tidx_hbm, nidx_hbm, emb_hbm,
                    outc_hbm, outt_hbm, outn_hbm,
                    idx_c, idx_t, idx_n, rows_c, rows_t, rows_n, sem):
    w = lax.axis_index("s") * NC + lax.axis_index("c")
    pltpu.sync_copy(cidx_hbm.at[pl.ds(w * BPW, BPW)], idx_c)
    pltpu.sync_copy(tidx_hbm.at[pl.ds(w * BPW, BPW)], idx_t)
    for j in range(NCHUNK):
        pltpu.sync_copy(
            nidx_hbm.at[pl.ds(w * NPW + j * 128, 128)], idx_n[j])
    cps = [
        pltpu.async_copy(emb_hbm.at[idx_c], rows_c, sem),
        pltpu.async_copy(emb_hbm.at[idx_t], rows_t, sem),
    ]
    for j in range(NCHUNK):
        cps.append(pltpu.async_copy(
            emb_hbm.at[idx_n[j]],
            rows_n.at[pl.ds(j * 128, 128)], sem))
    for cp in cps:
        cp.wait()
    pltpu.sync_copy(rows_c, outc_hbm.at[pl.ds(w * BPW, BPW)])
    pltpu.sync_copy(rows_t, outt_hbm.at[pl.ds(w * BPW, BPW)])
    pltpu.sync_copy(rows_n, outn_hbm.at[pl.ds(w * NPW, NPW)])


_sc_gather = functools.partial(
    pl.kernel,
    out_type=(
        jax.ShapeDtypeStruct((B, RW), jnp.float32),
        jax.ShapeDtypeStruct((B, RW), jnp.float32),
        jax.ShapeDtypeStruct((B * K, RW), jnp.float32),
    ),
    mesh=plsc.VectorSubcoreMesh(core_axis_name="c", subcore_axis_name="s"),
    compiler_params=pltpu.CompilerParams(use_tc_tiling_on_sc=True),
    scratch_types=[
        pltpu.VMEM((BPW,), jnp.int32),
        pltpu.VMEM((BPW,), jnp.int32),
        [pltpu.VMEM((128,), jnp.int32) for _ in range(NCHUNK)],
        pltpu.VMEM((BPW, RW), jnp.float32),
        pltpu.VMEM((BPW, RW), jnp.float32),
        pltpu.VMEM((NPW, RW), jnp.float32),
        pltpu.SemaphoreType.DMA,
    ],
)(_sc_gather_body)


def _tc_loss_body(c_ref, t_ref, n_ref, out_ref):
    c = c_ref[:, pl.ds(0, DIM)]
    t = t_ref[:, pl.ds(0, DIM)]
    nsum = jnp.zeros((B, DIM), jnp.float32)
    for k in range(K):
        nsum = nsum + n_ref[pl.ds(k * B, B), pl.ds(0, DIM)]
    p = jnp.sum(t * c, axis=1, keepdims=True)
    n = -jnp.sum(nsum * c, axis=1, keepdims=True)
    loss = jax.nn.log_sigmoid(p) + jax.nn.log_sigmoid(n)
    out_ref[0, 0] = -jnp.mean(loss)


def kernel(center, target, noise, embeddings):
    center = center.astype(jnp.int32)
    target = target.astype(jnp.int32)
    nidx = jnp.transpose(noise.astype(jnp.int32)).reshape(B * K)
    emb_p = _tc_transpose(jnp.transpose(embeddings))
    c_rows, t_rows, n_rows = _sc_gather(center, target, nidx, emb_p)
    out = pl.pallas_call(
        _tc_loss_body,
        out_shape=jax.ShapeDtypeStruct((1, 1), jnp.float32),
        out_specs=pl.BlockSpec(memory_space=pltpu.SMEM),
    )(c_rows, t_rows, n_rows)
    return out[0, 0]
